# baseline (speedup 1.0000x reference)
import functools

import jax
import jax.numpy as jnp
from jax import lax
from jax.experimental import pallas as pl
from jax.experimental.pallas import tpu as pltpu
from jax.experimental.pallas import tpu_sc as plsc

_LN_EPS = 1e-5
_NC = 2
_NS = 16
_NW = _NC * _NS
_IDXW = 128


def _repack_body(tt_ref, out_ref):
    x = tt_ref[...]
    xt = x.T
    out_ref[...] = jnp.concatenate([xt, xt], axis=1)


def _repack(table_t, blk):
    d, v = table_t.shape
    return pl.pallas_call(
        _repack_body,
        grid=(pl.cdiv(v, blk),),
        in_specs=[pl.BlockSpec((d, blk), lambda i: (0, i))],
        out_specs=pl.BlockSpec((blk, 2 * d), lambda i: (i, 0)),
        out_shape=jax.ShapeDtypeStruct((v, 2 * d), jnp.float32),
    )(table_t)


def _gather_body(nchunks, ids_hbm, table_hbm, out_hbm, idx_v, rows_v, sem):
    wid = lax.axis_index("s") * _NC + lax.axis_index("c")

    def step(i, carry):
        ci = wid * nchunks + i
        pltpu.sync_copy(ids_hbm.at[ci], idx_v)
        for half in range(2):
            copies = []
            for j in range(4):
                cp = pltpu.make_async_copy(
                    table_hbm.at[idx_v.at[half * 4 + j]],
                    rows_v.at[pl.ds(j * _IDXW, _IDXW)],
                    sem,
                )
                cp.start()
                copies.append(cp)
            for cp in copies:
                cp.wait()
            off = pl.multiple_of((ci * 2 + half) * 512, 512)
            pltpu.sync_copy(rows_v, out_hbm.at[pl.ds(off, 512)])
        return carry

    lax.fori_loop(0, nchunks, step, 0)


def _sc_gather(ids3d, table128):
    n = ids3d.shape[0] * 1024
    nchunks = ids3d.shape[0] // _NW
    mesh = plsc.VectorSubcoreMesh(core_axis_name="c", subcore_axis_name="s")
    f = pl.kernel(
        functools.partial(_gather_body, nchunks),
        out_type=jax.ShapeDtypeStruct((n, 128), jnp.float32),
        mesh=mesh,
        scratch_types=[
            pltpu.VMEM((8, _IDXW), jnp.int32),
            pltpu.VMEM((512, 128), jnp.float32),
            pltpu.SemaphoreType.DMA,
        ],
    )
    return f(ids3d, table128)


def _lin_ln_t_body(w_ref, b_ref, g_ref, be_ref, emb_ref, out_ref):
    x = emb_ref[...]
    w = w_ref[...]
    y = lax.dot_general(w, x, (((1,), (1,)), ((), ())),
                        preferred_element_type=jnp.float32)
    y = y + b_ref[...]
    m = jnp.mean(y, axis=0, keepdims=True)
    c = y - m
    v = jnp.mean(c * c, axis=0, keepdims=True)
    r = (c * lax.rsqrt(v + _LN_EPS)) * g_ref[...] + be_ref[...]
    out_ref[...] = r[None]


def _lin_ln_t(emb, w128, b, gamma, beta, fields, bsz, blk):
    d = w128.shape[0]
    nb = bsz // blk
    return pl.pallas_call(
        _lin_ln_t_body,
        grid=(fields, nb),
        in_specs=[
            pl.BlockSpec((d, 2 * d), lambda f, i: (0, 0)),
            pl.BlockSpec((d, 1), lambda f, i: (0, 0)),
            pl.BlockSpec((d, 1), lambda f, i: (0, 0)),
            pl.BlockSpec((d, 1), lambda f, i: (0, 0)),
            pl.BlockSpec((blk, 2 * d), lambda f, i: (f * nb + i, 0)),
        ],
        out_specs=pl.BlockSpec((1, d, blk), lambda f, i: (f, 0, i)),
        out_shape=jax.ShapeDtypeStruct((fields, d, bsz), jnp.float32),
    )(w128, b.reshape(d, 1), gamma.reshape(d, 1), beta.reshape(d, 1), emb)


def kernel(concept_ids, table, W, b, gamma, beta):
    bsz, fields = concept_ids.shape
    d = table.shape[1]
    n = bsz * fields

    table128 = _repack(table.T, blk=512)

    ids = concept_ids.T.reshape(n).astype(jnp.int32)
    ids3d = ids.reshape(n // 1024, 8, _IDXW)

    emb = _sc_gather(ids3d, table128)

    w128 = jnp.concatenate([W, jnp.zeros_like(W)], axis=1)
    out_t = _lin_ln_t(emb, w128, b, gamma, beta, fields, bsz, blk=2048)
    return out_t.transpose(2, 0, 1)
